# baseline (speedup 1.0000x reference)
import functools

import jax
import jax.numpy as jnp
from jax import lax
from jax.experimental import pallas as pl
from jax.experimental.pallas import tpu as pltpu
from jax.experimental.pallas import tpu_sc as plsc

DIM = 32
L = 16


def _iota():
    return lax.iota(jnp.int32, L)


@functools.lru_cache(maxsize=None)
def _make_transpose(nrows, nw):
    mesh = plsc.VectorSubcoreMesh(core_axis_name="c", subcore_axis_name="s")
    orows = nrows * DIM // 128
    nfull = nrows // 128
    tail_rows = (nrows - nfull * 128) * DIM // 128
    nloop = (nfull + nw - 1) // nw

    @functools.partial(
        pl.kernel,
        out_type=jax.ShapeDtypeStruct((orows, 128), jnp.float32),
        mesh=mesh,
        scratch_types=[
            pltpu.VMEM((4, DIM, 128), jnp.float32),
            pltpu.VMEM((4, DIM, 128), jnp.float32),
            pltpu.VMEM((tail_rows, 128), jnp.float32),
            [pltpu.SemaphoreType.DMA for _ in range(4)],
            [pltpu.SemaphoreType.DMA for _ in range(4)],
        ],
        compiler_params=pltpu.CompilerParams(use_tc_tiling_on_sc=True, needs_layout_passes=False),
    )
    def transpose(tab_t, tail_rm, out, ibuf, obuf, tbuf, isems, osems):
        wid = lax.axis_index("s") * mesh.num_cores + lax.axis_index("c")
        kmax = (nfull - 1 - wid) // nw
        rlo = _iota()
        rhi = rlo + L

        @pl.when(wid == 0)
        def _():
            pltpu.sync_copy(tail_rm, tbuf)
            pltpu.sync_copy(tbuf, out.at[pl.ds(nfull * DIM, tail_rows)])

        def blk_of(k):
            return wid + jnp.minimum(k, kmax) * nw

        def fire_in(k, p):
            pltpu.async_copy(
                tab_t.at[:, pl.ds(blk_of(k) * 128, 128)],
                ibuf.at[p], isems[p])

        def wait_in(p):
            pltpu.make_async_copy(
                tab_t.at[:, pl.ds(0, 128)], ibuf.at[p], isems[p]).wait()

        def trans(p):
            @pl.loop(0, DIM, unroll=4)
            def row(r):
                t0 = r * 4
                for j in range(4):
                    colv = jnp.zeros((L,), jnp.int32) + (t0 + j)
                    obuf[p, r, pl.ds(j * DIM, L)] = plsc.load_gather(
                        ibuf.at[p], [rlo, colv])
                    obuf[p, r, pl.ds(j * DIM + L, L)] = plsc.load_gather(
                        ibuf.at[p], [rhi, colv])

        def start_out(k, p):
            pltpu.async_copy(
                obuf.at[p], out.at[pl.ds(blk_of(k) * DIM, DIM)], osems[p])

        def wait_out(p):
            pltpu.make_async_copy(
                obuf.at[p], out.at[pl.ds(0, DIM)], osems[p]).wait()

        nss = (nloop + 3) // 4
        for s in range(4):
            fire_in(s, s)

        @pl.loop(0, nss)
        def superstep(u):
            for s in range(4):
                k = u * 4 + s

                @pl.when(u >= 1)
                def _(s=s):
                    wait_out(s)

                wait_in(s)
                trans(s)
                start_out(k, s)

                @pl.when(u < nss - 1)
                def _(k=k, s=s):
                    fire_in(k + 4, s)

        for s in range(4):
            wait_out(s)

    return transpose


@functools.lru_cache(maxsize=None)
def _make_gather(orows, bsz, hist, nw):
    mesh = plsc.VectorSubcoreMesh(core_axis_name="c", subcore_axis_name="s")
    bb_per_w = bsz // (128 * nw)
    half = hist // 2

    @functools.partial(
        pl.kernel,
        out_type=jax.ShapeDtypeStruct((hist, DIM, bsz), jnp.float32),
        mesh=mesh,
        scratch_types=[
            pltpu.VMEM((hist, 128), jnp.int32),
            pltpu.VMEM((hist, 128), jnp.int32),
            pltpu.VMEM((4, 128, 128), jnp.float32),
            pltpu.VMEM((4, DIM, 128), jnp.float32),
            [pltpu.SemaphoreType.DMA for _ in range(4)],
            [pltpu.SemaphoreType.DMA for _ in range(4)],
        ],
        compiler_params=pltpu.CompilerParams(use_tc_tiling_on_sc=True, needs_layout_passes=False),
    )
    def gather(tab, idx_t, out, idxb, ridb, gbuf, obuf, gsems, osems):
        wid = lax.axis_index("s") * mesh.num_cores + lax.axis_index("c")
        col0 = wid * bb_per_w * 128
        jvecs = [_iota() + g * L for g in range(128 // L)]
        nss = (hist + 3) // 4 + (1 if hist % 4 == 0 else 0)

        @pl.loop(0, bb_per_w)
        def bbloop(bb):
            colb = col0 + bb * 128
            pltpu.sync_copy(idx_t.at[:, pl.ds(colb, 128)], idxb)

            @pl.loop(0, hist)
            def prep(h):
                for g in range(128 // L):
                    ridb[h, pl.ds(g * L, L)] = (
                        idxb[h, pl.ds(g * L, L)] >> 2)

            def hk(k):
                return jnp.minimum(k, hist - 1)

            def fire(k, s):
                pltpu.async_copy(tab.at[ridb.at[hk(k)]], gbuf.at[s],
                                 gsems[s])

            def wait_gather(s):
                pltpu.make_async_copy(
                    tab.at[pl.ds(0, 128)], gbuf.at[s], gsems[s]).wait()

            def select(k, s):
                h = hk(k)
                for g in range(128 // L):
                    sub = (idxb[h, pl.ds(g * L, L)] & 3) * DIM
                    for c in range(DIM):
                        obuf[s, c, pl.ds(g * L, L)] = plsc.load_gather(
                            gbuf.at[s], [jvecs[g], sub + c])

            def start_out(k, s):
                pltpu.async_copy(
                    obuf.at[s],
                    out.at[hk(k), :, pl.ds(colb, 128)], osems[s])

            def wait_out(s):
                pltpu.make_async_copy(
                    obuf.at[s], out.at[0, :, pl.ds(0, 128)],
                    osems[s]).wait()

            for s in range(4):
                fire(s, s)

            @pl.loop(0, nss)
            def superstep(u):
                for s in range(4):
                    k = u * 4 + s

                    @pl.when(u >= 1)
                    def _(s=s):
                        wait_out(s)

                    wait_gather(s)
                    select(k, s)
                    start_out(k, s)

                    @pl.when(u < nss - 1)
                    def _(k=k, s=s):
                        fire(k + 4, s)

            for s in range(4):
                wait_out(s)

    return gather


def kernel(token_ids, embedding_matrix):
    bsz, hist = token_ids.shape
    nrows = embedding_matrix.shape[0]
    info = plsc.get_sparse_core_info()
    nw = info.num_cores * info.num_subcores
    table_t = embedding_matrix.T
    idx_t = token_ids.T.astype(jnp.int32)
    ntail = nrows - (nrows // 128) * 128
    tail_rm = embedding_matrix[nrows - ntail:, :].reshape(
        ntail * DIM // 128, 128)
    table_rm = _make_transpose(nrows, nw)(table_t, tail_rm)
    out_t = _make_gather(table_rm.shape[0], bsz, hist, nw)(table_rm, idx_t)
    return out_t.transpose(2, 0, 1)


# speedup vs baseline: 1.0761x; 1.0761x over previous
import functools

import jax
import jax.numpy as jnp
from jax import lax
from jax.experimental import pallas as pl
from jax.experimental.pallas import tpu as pltpu
from jax.experimental.pallas import tpu_sc as plsc

DIM = 32
L = 16


def _iota():
    return lax.iota(jnp.int32, L)


@functools.lru_cache(maxsize=None)
def _make_transpose(nrows, nw):
    mesh = plsc.VectorSubcoreMesh(core_axis_name="c", subcore_axis_name="s")
    orows = nrows * DIM // 128
    nfull = nrows // 128
    tail_rows = (nrows - nfull * 128) * DIM // 128
    nloop = (nfull + nw - 1) // nw

    @functools.partial(
        pl.kernel,
        out_type=jax.ShapeDtypeStruct((orows, 128), jnp.float32),
        mesh=mesh,
        scratch_types=[
            pltpu.VMEM((2, DIM, 128), jnp.float32),
            pltpu.VMEM((2, DIM, 128), jnp.float32),
            pltpu.VMEM((tail_rows, 128), jnp.float32),
            pltpu.SemaphoreType.DMA,
            pltpu.SemaphoreType.DMA,
            pltpu.SemaphoreType.DMA,
            pltpu.SemaphoreType.DMA,
        ],
        compiler_params=pltpu.CompilerParams(use_tc_tiling_on_sc=True, needs_layout_passes=False),
    )
    def transpose(tab_t, tail_rm, out, ibuf, obuf, tbuf, i0, i1, o0, o1):
        isems = (i0, i1)
        osems = (o0, o1)
        wid = lax.axis_index("s") * mesh.num_cores + lax.axis_index("c")
        kmax = (nfull - 1 - wid) // nw
        rlo = _iota()
        rhi = rlo + L

        @pl.when(wid == 0)
        def _():
            pltpu.sync_copy(tail_rm, tbuf)
            pltpu.sync_copy(tbuf, out.at[pl.ds(nfull * DIM, tail_rows)])

        def blk_of(k):
            return wid + jnp.minimum(k, kmax) * nw

        def fire_in(k, p):
            pltpu.async_copy(
                tab_t.at[:, pl.ds(blk_of(k) * 128, 128)],
                ibuf.at[p], isems[p])

        def wait_in(p):
            pltpu.make_async_copy(
                tab_t.at[:, pl.ds(0, 128)], ibuf.at[p], isems[p]).wait()

        def trans(p):
            @pl.loop(0, DIM)
            def row(r):
                t0 = r * 4
                for j in range(4):
                    colv = jnp.zeros((L,), jnp.int32) + (t0 + j)
                    obuf[p, r, pl.ds(j * DIM, L)] = plsc.load_gather(
                        ibuf.at[p], [rlo, colv])
                    obuf[p, r, pl.ds(j * DIM + L, L)] = plsc.load_gather(
                        ibuf.at[p], [rhi, colv])

        def start_out(k, p):
            pltpu.async_copy(
                obuf.at[p], out.at[pl.ds(blk_of(k) * DIM, DIM)], osems[p])

        def wait_out(p):
            pltpu.make_async_copy(
                obuf.at[p], out.at[pl.ds(0, DIM)], osems[p]).wait()

        half = (nloop + 1) // 2
        fire_in(0, 0)

        @pl.loop(0, half)
        def superstep(t):
            @pl.when(t >= 1)
            def _():
                wait_out(0)

            fire_in(2 * t + 1, 1)
            wait_in(0)
            trans(0)
            start_out(2 * t, 0)

            @pl.when(t < half - 1)
            def _():
                fire_in(2 * t + 2, 0)

            @pl.when(t >= 1)
            def _():
                wait_out(1)

            wait_in(1)
            trans(1)
            start_out(2 * t + 1, 1)

        wait_out(0)
        wait_out(1)

    return transpose


@functools.lru_cache(maxsize=None)
def _make_gather(orows, bsz, hist, nw):
    mesh = plsc.VectorSubcoreMesh(core_axis_name="c", subcore_axis_name="s")
    bb_per_w = bsz // (128 * nw)
    half = hist // 2

    @functools.partial(
        pl.kernel,
        out_type=jax.ShapeDtypeStruct((hist, DIM, bsz), jnp.float32),
        mesh=mesh,
        scratch_types=[
            [pltpu.VMEM((hist, 128), jnp.int32) for _ in range(bb_per_w)],
            [pltpu.VMEM((hist, 128), jnp.int32) for _ in range(bb_per_w)],
            pltpu.VMEM((2, 128, 128), jnp.float32),
            pltpu.VMEM((2, DIM, 128), jnp.float32),
            pltpu.SemaphoreType.DMA,
            pltpu.SemaphoreType.DMA,
            pltpu.SemaphoreType.DMA,
            pltpu.SemaphoreType.DMA,
        ],
        compiler_params=pltpu.CompilerParams(use_tc_tiling_on_sc=True, needs_layout_passes=False),
    )
    def gather(tab, idx_t, out, idxs, rids, gbuf, obuf, g0, g1, o0, o1):
        gsems = (g0, g1)
        osems = (o0, o1)
        wid = lax.axis_index("s") * mesh.num_cores + lax.axis_index("c")
        col0 = wid * bb_per_w * 128
        jvecs = [_iota() + g * L for g in range(128 // L)]

        for bb in range(bb_per_w):
            pltpu.sync_copy(idx_t.at[:, pl.ds(col0 + bb * 128, 128)],
                            idxs[bb])

            @pl.loop(0, hist)
            def prep(h, bb=bb):
                for g in range(128 // L):
                    rids[bb][h, pl.ds(g * L, L)] = (
                        idxs[bb][h, pl.ds(g * L, L)] >> 2)

        for bb in range(bb_per_w):
            def fire(h, p, bb=bb):
                pltpu.async_copy(tab.at[rids[bb].at[h]], gbuf.at[p],
                                 gsems[p])

            def wait_gather(p):
                pltpu.make_async_copy(
                    tab.at[pl.ds(0, 128)], gbuf.at[p], gsems[p]).wait()

            def select(h, p, bb=bb):
                for g in range(128 // L):
                    sub = (idxs[bb][h, pl.ds(g * L, L)] & 3) * DIM

                    @pl.loop(0, 4)
                    def cblk(ci, sub=sub, g=g):
                        for cj in range(8):
                            c = ci * 8 + cj
                            obuf[p, c, pl.ds(g * L, L)] = plsc.load_gather(
                                gbuf.at[p], [jvecs[g], sub + c])

            def start_out(h, p, bb=bb):
                pltpu.async_copy(
                    obuf.at[p],
                    out.at[h, :, pl.ds(col0 + bb * 128, 128)], osems[p])

            def wait_out(p):
                pltpu.make_async_copy(
                    obuf.at[p], out.at[0, :, pl.ds(0, 128)],
                    osems[p]).wait()

            fire(0, 0)

            @pl.loop(0, half)
            def superstep(t):
                @pl.when(t >= 1)
                def _():
                    wait_out(0)

                fire(2 * t + 1, 1)
                wait_gather(0)
                select(2 * t, 0)
                start_out(2 * t, 0)

                @pl.when(t < half - 1)
                def _():
                    fire(2 * t + 2, 0)

                @pl.when(t >= 1)
                def _():
                    wait_out(1)

                wait_gather(1)
                select(2 * t + 1, 1)
                start_out(2 * t + 1, 1)

            wait_out(0)
            wait_out(1)

    return gather


def kernel(token_ids, embedding_matrix):
    bsz, hist = token_ids.shape
    nrows = embedding_matrix.shape[0]
    info = plsc.get_sparse_core_info()
    nw = info.num_cores * info.num_subcores
    table_t = embedding_matrix.T
    idx_t = token_ids.T.astype(jnp.int32)
    ntail = nrows - (nrows // 128) * 128
    tail_rm = embedding_matrix[nrows - ntail:, :].reshape(
        ntail * DIM // 128, 128)
    table_rm = _make_transpose(nrows, nw)(table_t, tail_rm)
    out_t = _make_gather(table_rm.shape[0], bsz, hist, nw)(table_rm, idx_t)
    return out_t.transpose(2, 0, 1)


# speedup vs baseline: 1.6757x; 1.5571x over previous
import functools

import jax
import jax.numpy as jnp
from jax import lax
from jax.experimental import pallas as pl
from jax.experimental.pallas import tpu as pltpu
from jax.experimental.pallas import tpu_sc as plsc

DIM = 32
L = 16


def _iota():
    return lax.iota(jnp.int32, L)


@functools.lru_cache(maxsize=None)
def _make_transpose(nrows, nw):
    mesh = plsc.VectorSubcoreMesh(core_axis_name="c", subcore_axis_name="s")
    orows = nrows * DIM // 128
    nfull = nrows // 128
    tail_rows = (nrows - nfull * 128) * DIM // 128
    nloop = (nfull + nw - 1) // nw

    @functools.partial(
        pl.kernel,
        out_type=jax.ShapeDtypeStruct((orows, 128), jnp.float32),
        mesh=mesh,
        scratch_types=[
            pltpu.VMEM((2, DIM, 128), jnp.float32),
            pltpu.VMEM((2, DIM, 128), jnp.float32),
            pltpu.VMEM((tail_rows, 128), jnp.float32),
            pltpu.SemaphoreType.DMA,
            pltpu.SemaphoreType.DMA,
            pltpu.SemaphoreType.DMA,
            pltpu.SemaphoreType.DMA,
        ],
        compiler_params=pltpu.CompilerParams(use_tc_tiling_on_sc=True, needs_layout_passes=False),
    )
    def transpose(tab_t, tail_rm, out, ibuf, obuf, tbuf, i0, i1, o0, o1):
        isems = (i0, i1)
        osems = (o0, o1)
        wid = lax.axis_index("s") * mesh.num_cores + lax.axis_index("c")
        kmax = (nfull - 1 - wid) // nw
        rlo = _iota()
        rhi = rlo + L

        @pl.when(wid == 0)
        def _():
            pltpu.sync_copy(tail_rm, tbuf)
            pltpu.sync_copy(tbuf, out.at[pl.ds(nfull * DIM, tail_rows)])

        def blk_of(k):
            return wid + jnp.minimum(k, kmax) * nw

        def fire_in(k, p):
            pltpu.async_copy(
                tab_t.at[:, pl.ds(blk_of(k) * 128, 128)],
                ibuf.at[p], isems[p])

        def wait_in(p):
            pltpu.make_async_copy(
                tab_t.at[:, pl.ds(0, 128)], ibuf.at[p], isems[p]).wait()

        def trans(p):
            @plsc.parallel_loop(0, DIM)
            def row(r):
                t0 = r * 4
                for j in range(4):
                    colv = jnp.zeros((L,), jnp.int32) + (t0 + j)
                    obuf[p, r, pl.ds(j * DIM, L)] = plsc.load_gather(
                        ibuf.at[p], [rlo, colv])
                    obuf[p, r, pl.ds(j * DIM + L, L)] = plsc.load_gather(
                        ibuf.at[p], [rhi, colv])

        def start_out(k, p):
            pltpu.async_copy(
                obuf.at[p], out.at[pl.ds(blk_of(k) * DIM, DIM)], osems[p])

        def wait_out(p):
            pltpu.make_async_copy(
                obuf.at[p], out.at[pl.ds(0, DIM)], osems[p]).wait()

        half = (nloop + 1) // 2
        fire_in(0, 0)

        @pl.loop(0, half)
        def superstep(t):
            @pl.when(t >= 1)
            def _():
                wait_out(0)

            fire_in(2 * t + 1, 1)
            wait_in(0)
            trans(0)
            start_out(2 * t, 0)

            @pl.when(t < half - 1)
            def _():
                fire_in(2 * t + 2, 0)

            @pl.when(t >= 1)
            def _():
                wait_out(1)

            wait_in(1)
            trans(1)
            start_out(2 * t + 1, 1)

        wait_out(0)
        wait_out(1)

    return transpose


@functools.lru_cache(maxsize=None)
def _make_gather(orows, bsz, hist, nw):
    mesh = plsc.VectorSubcoreMesh(core_axis_name="c", subcore_axis_name="s")
    bb_per_w = bsz // (128 * nw)
    half = hist // 2

    @functools.partial(
        pl.kernel,
        out_type=jax.ShapeDtypeStruct((hist, DIM, bsz), jnp.float32),
        mesh=mesh,
        scratch_types=[
            [pltpu.VMEM((hist, 128), jnp.int32) for _ in range(bb_per_w)],
            [pltpu.VMEM((hist, 128), jnp.int32) for _ in range(bb_per_w)],
            pltpu.VMEM((2, 128, 128), jnp.float32),
            pltpu.VMEM((2, DIM, 128), jnp.float32),
            pltpu.SemaphoreType.DMA,
            pltpu.SemaphoreType.DMA,
            pltpu.SemaphoreType.DMA,
            pltpu.SemaphoreType.DMA,
        ],
        compiler_params=pltpu.CompilerParams(use_tc_tiling_on_sc=True, needs_layout_passes=False),
    )
    def gather(tab, idx_t, out, idxs, rids, gbuf, obuf, g0, g1, o0, o1):
        gsems = (g0, g1)
        osems = (o0, o1)
        wid = lax.axis_index("s") * mesh.num_cores + lax.axis_index("c")
        col0 = wid * bb_per_w * 128
        jvecs = [_iota() + g * L for g in range(128 // L)]

        for bb in range(bb_per_w):
            pltpu.sync_copy(idx_t.at[:, pl.ds(col0 + bb * 128, 128)],
                            idxs[bb])

            @pl.loop(0, hist)
            def prep(h, bb=bb):
                for g in range(128 // L):
                    rids[bb][h, pl.ds(g * L, L)] = (
                        idxs[bb][h, pl.ds(g * L, L)] >> 2)

        for bb in range(bb_per_w):
            def fire(h, p, bb=bb):
                pltpu.async_copy(tab.at[rids[bb].at[h]], gbuf.at[p],
                                 gsems[p])

            def wait_gather(p):
                pltpu.make_async_copy(
                    tab.at[pl.ds(0, 128)], gbuf.at[p], gsems[p]).wait()

            def select(h, p, bb=bb):
                for g in range(128 // L):
                    sub = (idxs[bb][h, pl.ds(g * L, L)] & 3) * DIM

                    @plsc.parallel_loop(0, 4)
                    def cblk(ci, sub=sub, g=g):
                        for cj in range(8):
                            c = ci * 8 + cj
                            obuf[p, c, pl.ds(g * L, L)] = plsc.load_gather(
                                gbuf.at[p], [jvecs[g], sub + c])

            def start_out(h, p, bb=bb):
                pltpu.async_copy(
                    obuf.at[p],
                    out.at[h, :, pl.ds(col0 + bb * 128, 128)], osems[p])

            def wait_out(p):
                pltpu.make_async_copy(
                    obuf.at[p], out.at[0, :, pl.ds(0, 128)],
                    osems[p]).wait()

            fire(0, 0)

            @pl.loop(0, half)
            def superstep(t):
                @pl.when(t >= 1)
                def _():
                    wait_out(0)

                fire(2 * t + 1, 1)
                wait_gather(0)
                select(2 * t, 0)
                start_out(2 * t, 0)

                @pl.when(t < half - 1)
                def _():
                    fire(2 * t + 2, 0)

                @pl.when(t >= 1)
                def _():
                    wait_out(1)

                wait_gather(1)
                select(2 * t + 1, 1)
                start_out(2 * t + 1, 1)

            wait_out(0)
            wait_out(1)

    return gather


def kernel(token_ids, embedding_matrix):
    bsz, hist = token_ids.shape
    nrows = embedding_matrix.shape[0]
    info = plsc.get_sparse_core_info()
    nw = info.num_cores * info.num_subcores
    table_t = embedding_matrix.T
    idx_t = token_ids.T.astype(jnp.int32)
    ntail = nrows - (nrows // 128) * 128
    tail_rm = embedding_matrix[nrows - ntail:, :].reshape(
        ntail * DIM // 128, 128)
    table_rm = _make_transpose(nrows, nw)(table_t, tail_rm)
    out_t = _make_gather(table_rm.shape[0], bsz, hist, nw)(table_rm, idx_t)
    return out_t.transpose(2, 0, 1)
